# Initial kernel scaffold; baseline (speedup 1.0000x reference)
import jax
import jax.numpy as jnp
from jax.experimental import pallas as pl


def kernel(x, exclusivities):
    raise NotImplementedError("write your pallas kernel here")


# speedup vs baseline: 2.3930x; 2.3930x over previous
import functools

import jax
import jax.numpy as jnp
from jax import lax
from jax.experimental import pallas as pl
from jax.experimental.pallas import tpu as pltpu
from jax.experimental.pallas import tpu_sc as plsc

_P = 1024
_C = 2 * _P
_R = 4 * 2048
_NPH = 256

_info = plsc.get_sparse_core_info()
_NC = _info.num_cores
_NS = _info.num_subcores
_L = _info.num_lanes
_NW = _NC * _NS

_ROWS_PER_W = _R // _NW
_BR = 16
_NBLK = _ROWS_PER_W // _BR


@functools.partial(
    pl.kernel,
    out_type=jax.ShapeDtypeStruct((_R, _C), jnp.float32),
    mesh=plsc.VectorSubcoreMesh(core_axis_name="c", subcore_axis_name="s"),
    compiler_params=pltpu.CompilerParams(needs_layout_passes=False),
    scratch_types=[
        pltpu.VMEM((_NPH,), jnp.int32),
        pltpu.VMEM((_BR, _C), jnp.float32),
    ],
)
def _sc_exclusivity(x_hbm, excl_hbm, out_hbm, excl_v, buf):
    wid = lax.axis_index("s") * _NC + lax.axis_index("c")
    base = wid * _ROWS_PER_W
    iota = lax.iota(jnp.int32, _L)
    even = (iota & 1) == 0

    pltpu.sync_copy(excl_hbm, excl_v)

    def chunk_body(c, r, rvec):
        tvec = c * _L + iota
        col = plsc.load_gather(excl_v, [tvec])
        prt = plsc.load_gather(excl_v, [tvec ^ 1])
        pv = plsc.load_gather(buf, [rvec, col + _P])
        pp = plsc.load_gather(buf, [rvec, prt + _P])
        sv = plsc.load_gather(buf, [rvec, col])
        keep = jnp.where(even, pv >= pp, pv > pp)
        plsc.store_scatter(buf, [rvec, col], jnp.where(keep, sv, jnp.zeros_like(sv)))

    def row_body(r, carry):
        rvec = jnp.full((_L,), r, dtype=jnp.int32)

        def cbody(c, carry2):
            chunk_body(c, r, rvec)
            return carry2

        lax.fori_loop(0, _NPH // _L, cbody, 0)
        return carry

    for g in range(_NBLK):
        row0 = base + g * _BR
        pltpu.sync_copy(x_hbm.at[pl.ds(row0, _BR)], buf)
        lax.fori_loop(0, _BR, row_body, 0)
        pltpu.sync_copy(buf, out_hbm.at[pl.ds(row0, _BR)])


def kernel(x, exclusivities):
    xf = x.reshape(_R, _C)
    ef = exclusivities.reshape(-1)
    out = _sc_exclusivity(xf, ef)
    return out.reshape(x.shape)


# speedup vs baseline: 3.7081x; 1.5496x over previous
import functools

import jax
import jax.numpy as jnp
from jax import lax
from jax.experimental import pallas as pl
from jax.experimental.pallas import tpu as pltpu
from jax.experimental.pallas import tpu_sc as plsc

_P = 1024
_C = 2 * _P
_R = 4 * 2048
_NPH = 256

_info = plsc.get_sparse_core_info()
_NC = _info.num_cores
_NS = _info.num_subcores
_L = _info.num_lanes
_NW = _NC * _NS

_ROWS_PER_W = _R // _NW
_BR = 16
_NBLK = _ROWS_PER_W // _BR


_NBUF = 3


@functools.partial(
    pl.kernel,
    out_type=jax.ShapeDtypeStruct((_R, _C), jnp.float32),
    mesh=plsc.VectorSubcoreMesh(core_axis_name="c", subcore_axis_name="s"),
    compiler_params=pltpu.CompilerParams(needs_layout_passes=False),
    scratch_types=[
        pltpu.VMEM((_NPH,), jnp.int32),
        [pltpu.VMEM((_BR, _C), jnp.float32) for _ in range(_NBUF)],
        [pltpu.SemaphoreType.DMA for _ in range(_NBUF)],
        [pltpu.SemaphoreType.DMA for _ in range(_NBUF)],
    ],
)
def _sc_exclusivity(x_hbm, excl_hbm, out_hbm, excl_v, bufs, sems_in, sems_out):
    wid = lax.axis_index("s") * _NC + lax.axis_index("c")
    base = wid * _ROWS_PER_W
    iota = lax.iota(jnp.int32, _L)
    even = (iota & 1) == 0

    pltpu.sync_copy(excl_hbm, excl_v)

    def compute_block(buf):
        def row_body(r, carry):
            rvec = jnp.full((_L,), r, dtype=jnp.int32)

            def cbody(c, carry2):
                tvec = c * _L + iota
                col = plsc.load_gather(excl_v, [tvec])
                prt = plsc.load_gather(excl_v, [tvec ^ 1])
                pv = plsc.load_gather(buf, [rvec, col + _P])
                pp = plsc.load_gather(buf, [rvec, prt + _P])
                sv = plsc.load_gather(buf, [rvec, col])
                keep = jnp.where(even, pv >= pp, pv > pp)
                plsc.store_scatter(
                    buf, [rvec, col], jnp.where(keep, sv, jnp.zeros_like(sv))
                )
                return carry2

            lax.fori_loop(0, _NPH // _L, cbody, 0)
            return carry

        lax.fori_loop(0, _BR, row_body, 0)

    def start_in(g):
        s = g % _NBUF
        return pltpu.async_copy(
            x_hbm.at[pl.ds(base + g * _BR, _BR)], bufs[s], sems_in[s]
        )

    def start_out(g):
        s = g % _NBUF
        return pltpu.async_copy(
            bufs[s], out_hbm.at[pl.ds(base + g * _BR, _BR)], sems_out[s]
        )

    in_h = {g: start_in(g) for g in range(min(2, _NBLK))}
    out_h = {}
    for g in range(_NBLK):
        in_h[g].wait()
        compute_block(bufs[g % _NBUF])
        out_h[g] = start_out(g)
        nxt = g + 2
        if nxt < _NBLK:
            if g - 1 >= 0:
                out_h[g - 1].wait()
            in_h[nxt] = start_in(nxt)
    for g in range(max(0, _NBLK - 3), _NBLK):
        out_h[g].wait()


def kernel(x, exclusivities):
    xf = x.reshape(_R, _C)
    ef = exclusivities.reshape(-1)
    out = _sc_exclusivity(xf, ef)
    return out.reshape(x.shape)


# speedup vs baseline: 4.5757x; 1.2340x over previous
import functools

import jax
import jax.numpy as jnp
from jax import lax
from jax.experimental import pallas as pl
from jax.experimental.pallas import tpu as pltpu
from jax.experimental.pallas import tpu_sc as plsc

_P = 1024
_C = 2 * _P
_R = 4 * 2048
_NPH = 256

_info = plsc.get_sparse_core_info()
_NC = _info.num_cores
_NS = _info.num_subcores
_L = _info.num_lanes
_NW = _NC * _NS

_ROWS_PER_W = _R // _NW
_BR = 16
_NBLK = _ROWS_PER_W // _BR


_NBUF = 3
_RUN = 4


@functools.partial(
    pl.kernel,
    out_type=jax.ShapeDtypeStruct((_R, _C), jnp.float32),
    mesh=plsc.VectorSubcoreMesh(core_axis_name="c", subcore_axis_name="s"),
    compiler_params=pltpu.CompilerParams(needs_layout_passes=False),
    scratch_types=[
        pltpu.VMEM((_NPH,), jnp.int32),
        [pltpu.VMEM((_BR, _C), jnp.float32) for _ in range(_NBUF)],
        [pltpu.SemaphoreType.DMA for _ in range(_NBUF)],
        [pltpu.SemaphoreType.DMA for _ in range(_NBUF)],
    ],
)
def _sc_exclusivity(x_hbm, excl_hbm, out_hbm, excl_v, bufs, sems_in, sems_out):
    wid = lax.axis_index("s") * _NC + lax.axis_index("c")
    base = wid * _ROWS_PER_W
    iota = lax.iota(jnp.int32, _L)
    even = (iota & 1) == 0

    pltpu.sync_copy(excl_hbm, excl_v)

    def compute_block(buf):
        def chunk_body(kc, carry):
            t0 = (kc * _L + iota) * 2
            e0 = plsc.load_gather(excl_v, [t0])
            e1 = plsc.load_gather(excl_v, [t0 + 1])
            e0p = e0 + _P
            e1p = e1 + _P

            def row_body(rq, carry2):
                for j in range(_RUN):
                    rvec = jnp.full((_L,), rq * _RUN + j, dtype=jnp.int32)
                    a = plsc.load_gather(buf, [rvec, e0p])
                    b = plsc.load_gather(buf, [rvec, e1p])
                    s0 = plsc.load_gather(buf, [rvec, e0])
                    s1 = plsc.load_gather(buf, [rvec, e1])
                    plsc.store_scatter(buf, [rvec, e0], jnp.where(a >= b, s0, 0.0))
                    plsc.store_scatter(buf, [rvec, e1], jnp.where(b > a, s1, 0.0))
                return carry2

            lax.fori_loop(0, _BR // _RUN, row_body, 0)
            return carry

        lax.fori_loop(0, _NPH // (2 * _L), chunk_body, 0)

    def start_in(g):
        s = g % _NBUF
        return pltpu.async_copy(
            x_hbm.at[pl.ds(base + g * _BR, _BR)], bufs[s], sems_in[s]
        )

    def start_out(g):
        s = g % _NBUF
        return pltpu.async_copy(
            bufs[s], out_hbm.at[pl.ds(base + g * _BR, _BR)], sems_out[s]
        )

    in_h = {g: start_in(g) for g in range(min(2, _NBLK))}
    out_h = {}
    for g in range(_NBLK):
        in_h[g].wait()
        compute_block(bufs[g % _NBUF])
        out_h[g] = start_out(g)
        nxt = g + 2
        if nxt < _NBLK:
            if g - 1 >= 0:
                out_h[g - 1].wait()
            in_h[nxt] = start_in(nxt)
    for g in range(max(0, _NBLK - 3), _NBLK):
        out_h[g].wait()


def kernel(x, exclusivities):
    xf = x.reshape(_R, _C)
    ef = exclusivities.reshape(-1)
    out = _sc_exclusivity(xf, ef)
    return out.reshape(x.shape)
